# baseline (speedup 1.0000x reference)
import functools

import jax
import jax.numpy as jnp
from jax import lax
from jax.experimental import pallas as pl
from jax.experimental.pallas import tpu as pltpu
from jax.experimental.pallas import tpu_sc as plsc

N = 10000
E = 320000
D = 128
H = 128
NB = 20000
NA = 20000
ND = 30000
DEPTH = 3

NUM_SC = 2
NUM_SUBCORES = 16
NW = NUM_SC * NUM_SUBCORES
CH = 128
NBUF = 4
NBUF_E = 4

NP = 10240
TRASH_ROWS = NP - N

CH_E = 64
CPW_E = 160
PH_E = 8
PCH_E = CPW_E // PH_E
E_PAD = NW * CPW_E * CH_E

GA_REAL = 2 * NB + 3 * NA
GB_REAL = 4 * ND
NBUF_G = 5
CPW_GA = 25
CPW_GB = 30
GA_PAD = NW * CPW_GA * CH
GB_PAD = NW * CPW_GB * CH

ROWS_PER_TILE = NP // NUM_SUBCORES

_f32 = jnp.float32
_bf16 = jnp.bfloat16


def _vector_mesh():
    return plsc.VectorSubcoreMesh(core_axis_name="core", subcore_axis_name="subcore")


def _sc_round_agg(h, src3, dst3, zeros):
    @functools.partial(
        pl.kernel,
        out_type=jax.ShapeDtypeStruct((NUM_SC, NP, H), _f32),
        mesh=_vector_mesh(),
        scratch_types=[
            pltpu.VMEM((2, PCH_E, CH_E), jnp.int32),
            pltpu.VMEM((2, PCH_E, CH_E), jnp.int32),
            [pltpu.VMEM((CH_E, H), _f32) for _ in range(NBUF_E)],
            pltpu.VMEM_SHARED((NP, H), _f32),
            pltpu.SemaphoreType.DMA((NBUF_E,)),
            pltpu.SemaphoreType.DMA((NBUF_E,)),
            pltpu.SemaphoreType.DMA((2,)),
            pltpu.SemaphoreType.DMA,
        ],
    )
    def k(h_hbm, src_hbm, dst_hbm, z_hbm, out_hbm,
          src_v, dst_v, rows_v, agg_sh, sem_g, sem_s, sem_i, sem):
        cid = lax.axis_index("core")
        sid = lax.axis_index("subcore")
        wid = sid * NUM_SC + cid
        row0 = sid * ROWS_PER_TILE

        zc = pltpu.async_copy(
            z_hbm.at[pl.ds(row0, ROWS_PER_TILE)],
            agg_sh.at[pl.ds(row0, ROWS_PER_TILE)], sem)
        pltpu.async_copy(src_hbm.at[wid, 0], src_v.at[0], sem_i.at[0])
        pltpu.async_copy(dst_hbm.at[wid, 0], dst_v.at[0], sem_i.at[0])
        zc.wait()
        plsc.subcore_barrier()

        for p in range(PH_E):
            cur = p % 2
            pltpu.make_async_copy(
                src_hbm.at[wid, p], src_v.at[cur], sem_i.at[cur]).wait()
            pltpu.make_async_copy(
                dst_hbm.at[wid, p], dst_v.at[cur], sem_i.at[cur]).wait()
            if p + 1 < PH_E:
                nxt = (p + 1) % 2
                pltpu.async_copy(src_hbm.at[wid, p + 1], src_v.at[nxt],
                                 sem_i.at[nxt])
                pltpu.async_copy(dst_hbm.at[wid, p + 1], dst_v.at[nxt],
                                 sem_i.at[nxt])
            sv = src_v.at[cur]
            dv = dst_v.at[cur]

            for b in range(NBUF_E):
                pltpu.async_copy(h_hbm.at[sv.at[b]], rows_v[b], sem_g.at[b])

            @pl.loop(0, PCH_E, step=NBUF_E)
            def _(jj):
                for b in range(NBUF_E):
                    j = jj + b
                    pltpu.make_async_copy(
                        h_hbm.at[sv.at[j]], rows_v[b], sem_g.at[b]).wait()
                    pltpu.async_copy(
                        rows_v[b], agg_sh.at[dv.at[j]], sem_s.at[b], add=True)

                    @pl.when(j + NBUF_E < PCH_E)
                    def _():
                        pltpu.make_async_copy(
                            rows_v[b], agg_sh.at[dv.at[j]], sem_s.at[b]).wait()
                        pltpu.async_copy(
                            h_hbm.at[sv.at[j + NBUF_E]], rows_v[b], sem_g.at[b])

            for b in range(NBUF_E):
                j = PCH_E - NBUF_E + b
                pltpu.make_async_copy(
                    rows_v[b], agg_sh.at[dv.at[j]], sem_s.at[b]).wait()

        plsc.subcore_barrier()
        pltpu.async_copy(
            agg_sh.at[pl.ds(row0, ROWS_PER_TILE)],
            out_hbm.at[cid, pl.ds(row0, ROWS_PER_TILE)], sem).wait()

    return k(h, src3, dst3, zeros)


def _sc_term_gather(enc, idx3, cpw, out_rows):
    @functools.partial(
        pl.kernel,
        out_type=jax.ShapeDtypeStruct((out_rows, H), _f32),
        mesh=_vector_mesh(),
        scratch_types=[
            pltpu.VMEM((cpw, CH), jnp.int32),
            [pltpu.VMEM((CH, H), _f32) for _ in range(NBUF_G)],
            pltpu.SemaphoreType.DMA((NBUF_G,)),
            pltpu.SemaphoreType.DMA((NBUF_G,)),
            pltpu.SemaphoreType.DMA,
        ],
    )
    def k(enc_hbm, idx_hbm, out_hbm, idx_v, rows_v, sem_g, sem_s, sem):
        cid = lax.axis_index("core")
        sid = lax.axis_index("subcore")
        wid = sid * NUM_SC + cid
        base = wid * cpw * CH

        pltpu.async_copy(idx_hbm.at[wid], idx_v, sem).wait()

        for b in range(NBUF_G):
            pltpu.async_copy(enc_hbm.at[idx_v.at[b]], rows_v[b], sem_g.at[b])

        @pl.loop(0, cpw, step=NBUF_G)
        def _(jj):
            for b in range(NBUF_G):
                j = jj + b
                pltpu.make_async_copy(
                    enc_hbm.at[idx_v.at[j]], rows_v[b], sem_g.at[b]).wait()
                dst = out_hbm.at[pl.ds(base + j * CH, CH)]
                pltpu.async_copy(rows_v[b], dst, sem_s.at[b])

                @pl.when(j + NBUF_G < cpw)
                def _():
                    pltpu.make_async_copy(rows_v[b], dst, sem_s.at[b]).wait()
                    pltpu.async_copy(
                        enc_hbm.at[idx_v.at[j + NBUF_G]], rows_v[b], sem_g.at[b])

        for b in range(NBUF_G):
            j = cpw - NBUF_G + b
            dst = out_hbm.at[pl.ds(base + j * CH, CH)]
            pltpu.make_async_copy(rows_v[b], dst, sem_s.at[b]).wait()

    return k(enc, idx3)


_RB = 1000
_RBN = 1024


def _tc_encode_init(x, w_in):
    def body(x_ref, w_ref, o_ref):
        o_ref[...] = jnp.maximum(
            jnp.dot(x_ref[...], w_ref[...], preferred_element_type=_f32), 0.0)

    return pl.pallas_call(
        body,
        grid=(NP // _RBN,),
        in_specs=[
            pl.BlockSpec((_RBN, D), lambda i: (i, 0)),
            pl.BlockSpec((D, H), lambda i: (0, 0)),
        ],
        out_specs=pl.BlockSpec((_RBN, H), lambda i: (i, 0)),
        out_shape=jax.ShapeDtypeStruct((NP, H), _f32),
    )(x, w_in)


def _tc_round_update(h, parts, w_h):
    def body(h_ref, p_ref, w_ref, o_ref):
        agg = p_ref[0] + p_ref[1]
        o_ref[...] = jnp.maximum(
            h_ref[...] + jnp.dot(agg, w_ref[...], preferred_element_type=_f32),
            0.0)

    return pl.pallas_call(
        body,
        grid=(NP // _RBN,),
        in_specs=[
            pl.BlockSpec((_RBN, H), lambda i: (i, 0)),
            pl.BlockSpec((NUM_SC, _RBN, H), lambda i: (0, i, 0)),
            pl.BlockSpec((H, H), lambda i: (0, 0)),
        ],
        out_specs=pl.BlockSpec((_RBN, H), lambda i: (i, 0)),
        out_shape=jax.ShapeDtypeStruct((NP, H), _f32),
    )(h, parts, w_h)


def _tc_bond_mlp(ga, w1, b1, w2, b2):
    nblk = NB // _RB

    def body(g0_ref, g1_ref, w1_ref, b1_ref, w2_ref, b2_ref, o_ref):
        e = (g0_ref[...].astype(_f32) + g1_ref[...].astype(_f32)).astype(_bf16)
        t = jnp.maximum(
            jnp.dot(e, w1_ref[...], preferred_element_type=_f32) + b1_ref[...],
            0.0)
        o_ref[...] = jnp.dot(t, w2_ref[...], preferred_element_type=_f32) + b2_ref[...]

    return pl.pallas_call(
        body,
        grid=(nblk,),
        in_specs=[
            pl.BlockSpec((_RB, H), lambda i: (i, 0)),
            pl.BlockSpec((_RB, H), lambda i: (i + nblk, 0)),
            pl.BlockSpec((H, H), lambda i: (0, 0)),
            pl.BlockSpec((1, H), lambda i: (0, 0)),
            pl.BlockSpec((H, 2), lambda i: (0, 0)),
            pl.BlockSpec((1, 2), lambda i: (0, 0)),
        ],
        out_specs=pl.BlockSpec((_RB, 2), lambda i: (i, 0)),
        out_shape=jax.ShapeDtypeStruct((NB, 2), _f32),
    )(ga, ga, w1, b1, w2, b2)


def _tc_angle_mlp(ga, w1, b1, w2, b2):
    nblk = NA // _RB
    off = 2 * (NB // _RB)

    def body(e0_ref, e2_ref, c_ref, w1_ref, b1_ref, w2_ref, b2_ref, o_ref):
        ends = (e0_ref[...].astype(_f32) + e2_ref[...].astype(_f32)).astype(_bf16)
        t = (jnp.dot(ends, w1_ref[0:H, :], preferred_element_type=_f32)
             + jnp.dot(c_ref[...].astype(_bf16), w1_ref[H:2 * H, :],
                       preferred_element_type=_f32)
             + b1_ref[...])
        t = jnp.maximum(t, 0.0)
        o_ref[...] = jnp.dot(t, w2_ref[...], preferred_element_type=_f32) + b2_ref[...]

    return pl.pallas_call(
        body,
        grid=(nblk,),
        in_specs=[
            pl.BlockSpec((_RB, H), lambda i: (i + off, 0)),
            pl.BlockSpec((_RB, H), lambda i: (i + off + nblk, 0)),
            pl.BlockSpec((_RB, H), lambda i: (i + off + 2 * nblk, 0)),
            pl.BlockSpec((2 * H, H), lambda i: (0, 0)),
            pl.BlockSpec((1, H), lambda i: (0, 0)),
            pl.BlockSpec((H, 2), lambda i: (0, 0)),
            pl.BlockSpec((1, 2), lambda i: (0, 0)),
        ],
        out_specs=pl.BlockSpec((_RB, 2), lambda i: (i, 0)),
        out_shape=jax.ShapeDtypeStruct((NA, 2), _f32),
    )(ga, ga, ga, w1, b1, w2, b2)


def _tc_dihedral_mlp(ga, w1, w1r, b1, w2, b2):
    nblk = ND // _RB
    off = 0

    def body(g0, g1, g2, g3, w1_ref, w1r_ref, b1_ref, w2_ref, b2_ref, o_ref):
        gs = (g0[...].astype(_bf16), g1[...].astype(_bf16),
              g2[...].astype(_bf16), g3[...].astype(_bf16))
        t1 = b1_ref[...]
        t2 = b1_ref[...]
        for kk in range(4):
            blk = slice(kk * H, (kk + 1) * H)
            t1 = t1 + jnp.dot(gs[kk], w1_ref[blk, :], preferred_element_type=_f32)
            t2 = t2 + jnp.dot(gs[kk], w1r_ref[blk, :], preferred_element_type=_f32)
        t = jnp.maximum(t1, 0.0) + jnp.maximum(t2, 0.0)
        o_ref[...] = (0.5 * jnp.dot(t, w2_ref[...], preferred_element_type=_f32)
                      + b2_ref[...])

    return pl.pallas_call(
        body,
        grid=(nblk,),
        in_specs=[
            pl.BlockSpec((_RB, H), lambda i: (i + off, 0)),
            pl.BlockSpec((_RB, H), lambda i: (i + off + nblk, 0)),
            pl.BlockSpec((_RB, H), lambda i: (i + off + 2 * nblk, 0)),
            pl.BlockSpec((_RB, H), lambda i: (i + off + 3 * nblk, 0)),
            pl.BlockSpec((4 * H, H), lambda i: (0, 0)),
            pl.BlockSpec((4 * H, H), lambda i: (0, 0)),
            pl.BlockSpec((1, H), lambda i: (0, 0)),
            pl.BlockSpec((H, 4), lambda i: (0, 0)),
            pl.BlockSpec((1, 4), lambda i: (0, 0)),
        ],
        out_specs=pl.BlockSpec((_RB, 4), lambda i: (i, 0)),
        out_shape=jax.ShapeDtypeStruct((ND, 4), _f32),
    )(ga, ga, ga, ga, w1, w1r, b1, w2, b2)


def _tc_pair_mlp(enc, lj, w1, b1, w2, b2):
    def body(e_ref, lj_ref, w1_ref, b1_ref, w2_ref, b2_ref, o_ref):
        t = jnp.maximum(
            jnp.dot(e_ref[...], w1_ref[...], preferred_element_type=_f32)
            + b1_ref[...], 0.0)
        tp = jnp.dot(t, w2_ref[...], preferred_element_type=_f32) + b2_ref[...]
        o_ref[...] = jnp.concatenate([tp, lj_ref[...]], axis=1)

    return pl.pallas_call(
        body,
        grid=(N // _RB,),
        in_specs=[
            pl.BlockSpec((_RB, H), lambda i: (i, 0)),
            pl.BlockSpec((_RB, 2), lambda i: (i, 0)),
            pl.BlockSpec((H, H), lambda i: (0, 0)),
            pl.BlockSpec((1, H), lambda i: (0, 0)),
            pl.BlockSpec((H, 2), lambda i: (0, 0)),
            pl.BlockSpec((1, 2), lambda i: (0, 0)),
        ],
        out_specs=pl.BlockSpec((_RB, 4), lambda i: (i, 0)),
        out_shape=jax.ShapeDtypeStruct((N, 4), _f32),
    )(enc, lj, w1, b1, w2, b2)


def kernel(x, edge_index, bonds, angles, dihedrals, lj_params,
           W_in, W_h,
           bw1, bb1, bw2, bb2,
           aw1, ab1, aw2, ab2,
           dw1, db1, dw2, db2,
           pw1, pb1, pw2, pb2):
    src = edge_index[0]
    dst = edge_index[1]
    pad_e = E_PAD - E
    src_pad = (jnp.arange(pad_e, dtype=jnp.int32) * 37) % N
    dst_pad = N + (jnp.arange(pad_e, dtype=jnp.int32) % TRASH_ROWS)
    src3 = jnp.concatenate([src, src_pad]).reshape(NW, PH_E, PCH_E, CH_E)
    dst3 = jnp.concatenate([dst, dst_pad]).reshape(NW, PH_E, PCH_E, CH_E)
    ga_pad = (jnp.arange(GA_PAD - GA_REAL, dtype=jnp.int32) * 37) % N
    idxA3 = jnp.concatenate(
        [bonds[:, 0], bonds[:, 1],
         angles[:, 0], angles[:, 2], angles[:, 1],
         ga_pad]).reshape(NW, CPW_GA, CH)
    gb_pad = (jnp.arange(GB_PAD - GB_REAL, dtype=jnp.int32) * 37) % N
    idxB3 = jnp.concatenate(
        [dihedrals[:, 0], dihedrals[:, 1], dihedrals[:, 2], dihedrals[:, 3],
         gb_pad]).reshape(NW, CPW_GB, CH)
    zeros = jnp.zeros((NP, H), _f32)
    dw1b = dw1.astype(_bf16)
    dw1r = jnp.concatenate(
        [dw1b[3 * H:4 * H], dw1b[2 * H:3 * H], dw1b[H:2 * H], dw1b[0:H]], axis=0)

    xp = jnp.pad(x, ((0, NP - N), (0, 0)))
    h = _tc_encode_init(xp, W_in)
    for _ in range(DEPTH):
        parts = _sc_round_agg(h, src3, dst3, zeros)
        h = _tc_round_update(h, parts, W_h)

    gaA = _sc_term_gather(h, idxA3, CPW_GA, GA_PAD)
    gaB = _sc_term_gather(h, idxB3, CPW_GB, GB_PAD)

    bond_params = _tc_bond_mlp(
        gaA, bw1.astype(_bf16), bb1.reshape(1, H), bw2, bb2.reshape(1, 2))
    angle_params = _tc_angle_mlp(
        gaA, aw1.astype(_bf16), ab1.reshape(1, H), aw2, ab2.reshape(1, 2))
    pair_params = _tc_pair_mlp(
        h, lj_params, pw1, pb1.reshape(1, H), pw2, pb2.reshape(1, 2))
    dihedral_params = _tc_dihedral_mlp(
        gaB, dw1b, dw1r, db1.reshape(1, H), dw2, db2.reshape(1, 4))
    return bond_params, angle_params, dihedral_params, pair_params


# speedup vs baseline: 1.0155x; 1.0155x over previous
import functools

import jax
import jax.numpy as jnp
from jax import lax
from jax.experimental import pallas as pl
from jax.experimental.pallas import tpu as pltpu
from jax.experimental.pallas import tpu_sc as plsc

N = 10000
E = 320000
D = 128
H = 128
NB = 20000
NA = 20000
ND = 30000
DEPTH = 3

NUM_SC = 2
NUM_SUBCORES = 16
NW = NUM_SC * NUM_SUBCORES
CH = 128
NBUF = 4
NBUF_E = 3

NP = 10240
TRASH_ROWS = NP - N

CH_E = 96
CPW_E = 105
PH_E = 5
PCH_E = CPW_E // PH_E
E_PAD = NW * CPW_E * CH_E

GA_REAL = 2 * NB + 3 * NA
GB_REAL = 4 * ND
NBUF_G = 5
CPW_GA = 25
CPW_GB = 30
GA_PAD = NW * CPW_GA * CH
GB_PAD = NW * CPW_GB * CH

ROWS_PER_TILE = NP // NUM_SUBCORES

_f32 = jnp.float32
_bf16 = jnp.bfloat16


def _vector_mesh():
    return plsc.VectorSubcoreMesh(core_axis_name="core", subcore_axis_name="subcore")


def _sc_round_agg(h, src3, dst3, zeros):
    @functools.partial(
        pl.kernel,
        out_type=jax.ShapeDtypeStruct((NUM_SC, NP, H), _f32),
        mesh=_vector_mesh(),
        scratch_types=[
            pltpu.VMEM((2, PCH_E, CH_E), jnp.int32),
            pltpu.VMEM((2, PCH_E, CH_E), jnp.int32),
            [pltpu.VMEM((CH_E, H), _f32) for _ in range(NBUF_E)],
            pltpu.VMEM_SHARED((NP, H), _f32),
            pltpu.SemaphoreType.DMA((NBUF_E,)),
            pltpu.SemaphoreType.DMA((NBUF_E,)),
            pltpu.SemaphoreType.DMA((2,)),
            pltpu.SemaphoreType.DMA,
        ],
    )
    def k(h_hbm, src_hbm, dst_hbm, z_hbm, out_hbm,
          src_v, dst_v, rows_v, agg_sh, sem_g, sem_s, sem_i, sem):
        cid = lax.axis_index("core")
        sid = lax.axis_index("subcore")
        wid = sid * NUM_SC + cid
        row0 = sid * ROWS_PER_TILE

        zc = pltpu.async_copy(
            z_hbm.at[pl.ds(row0, ROWS_PER_TILE)],
            agg_sh.at[pl.ds(row0, ROWS_PER_TILE)], sem)
        pltpu.async_copy(src_hbm.at[wid, 0], src_v.at[0], sem_i.at[0])
        pltpu.async_copy(dst_hbm.at[wid, 0], dst_v.at[0], sem_i.at[0])
        zc.wait()
        plsc.subcore_barrier()

        for p in range(PH_E):
            cur = p % 2
            pltpu.make_async_copy(
                src_hbm.at[wid, p], src_v.at[cur], sem_i.at[cur]).wait()
            pltpu.make_async_copy(
                dst_hbm.at[wid, p], dst_v.at[cur], sem_i.at[cur]).wait()
            if p + 1 < PH_E:
                nxt = (p + 1) % 2
                pltpu.async_copy(src_hbm.at[wid, p + 1], src_v.at[nxt],
                                 sem_i.at[nxt])
                pltpu.async_copy(dst_hbm.at[wid, p + 1], dst_v.at[nxt],
                                 sem_i.at[nxt])
            sv = src_v.at[cur]
            dv = dst_v.at[cur]

            for b in range(NBUF_E):
                pltpu.async_copy(h_hbm.at[sv.at[b]], rows_v[b], sem_g.at[b])

            @pl.loop(0, PCH_E, step=NBUF_E)
            def _(jj):
                for b in range(NBUF_E):
                    j = jj + b
                    pltpu.make_async_copy(
                        h_hbm.at[sv.at[j]], rows_v[b], sem_g.at[b]).wait()
                    pltpu.async_copy(
                        rows_v[b], agg_sh.at[dv.at[j]], sem_s.at[b], add=True)

                    @pl.when(j + NBUF_E < PCH_E)
                    def _():
                        pltpu.make_async_copy(
                            rows_v[b], agg_sh.at[dv.at[j]], sem_s.at[b]).wait()
                        pltpu.async_copy(
                            h_hbm.at[sv.at[j + NBUF_E]], rows_v[b], sem_g.at[b])

            for b in range(NBUF_E):
                j = PCH_E - NBUF_E + b
                pltpu.make_async_copy(
                    rows_v[b], agg_sh.at[dv.at[j]], sem_s.at[b]).wait()

        plsc.subcore_barrier()
        pltpu.async_copy(
            agg_sh.at[pl.ds(row0, ROWS_PER_TILE)],
            out_hbm.at[cid, pl.ds(row0, ROWS_PER_TILE)], sem).wait()

    return k(h, src3, dst3, zeros)


def _sc_term_gather(enc, idx3, cpw, out_rows):
    @functools.partial(
        pl.kernel,
        out_type=jax.ShapeDtypeStruct((out_rows, H), _f32),
        mesh=_vector_mesh(),
        scratch_types=[
            pltpu.VMEM((cpw, CH), jnp.int32),
            [pltpu.VMEM((CH, H), _f32) for _ in range(NBUF_G)],
            pltpu.SemaphoreType.DMA((NBUF_G,)),
            pltpu.SemaphoreType.DMA((NBUF_G,)),
            pltpu.SemaphoreType.DMA,
        ],
    )
    def k(enc_hbm, idx_hbm, out_hbm, idx_v, rows_v, sem_g, sem_s, sem):
        cid = lax.axis_index("core")
        sid = lax.axis_index("subcore")
        wid = sid * NUM_SC + cid
        base = wid * cpw * CH

        pltpu.async_copy(idx_hbm.at[wid], idx_v, sem).wait()

        for b in range(NBUF_G):
            pltpu.async_copy(enc_hbm.at[idx_v.at[b]], rows_v[b], sem_g.at[b])

        @pl.loop(0, cpw, step=NBUF_G)
        def _(jj):
            for b in range(NBUF_G):
                j = jj + b
                pltpu.make_async_copy(
                    enc_hbm.at[idx_v.at[j]], rows_v[b], sem_g.at[b]).wait()
                dst = out_hbm.at[pl.ds(base + j * CH, CH)]
                pltpu.async_copy(rows_v[b], dst, sem_s.at[b])

                @pl.when(j + NBUF_G < cpw)
                def _():
                    pltpu.make_async_copy(rows_v[b], dst, sem_s.at[b]).wait()
                    pltpu.async_copy(
                        enc_hbm.at[idx_v.at[j + NBUF_G]], rows_v[b], sem_g.at[b])

        for b in range(NBUF_G):
            j = cpw - NBUF_G + b
            dst = out_hbm.at[pl.ds(base + j * CH, CH)]
            pltpu.make_async_copy(rows_v[b], dst, sem_s.at[b]).wait()

    return k(enc, idx3)


_RB = 1000
_RBN = 1024


def _tc_encode_init(x, w_in):
    def body(x_ref, w_ref, o_ref):
        o_ref[...] = jnp.maximum(
            jnp.dot(x_ref[...], w_ref[...], preferred_element_type=_f32), 0.0)

    return pl.pallas_call(
        body,
        grid=(NP // _RBN,),
        in_specs=[
            pl.BlockSpec((_RBN, D), lambda i: (i, 0)),
            pl.BlockSpec((D, H), lambda i: (0, 0)),
        ],
        out_specs=pl.BlockSpec((_RBN, H), lambda i: (i, 0)),
        out_shape=jax.ShapeDtypeStruct((NP, H), _f32),
    )(x, w_in)


def _tc_round_update(h, parts, w_h):
    def body(h_ref, p_ref, w_ref, o_ref):
        agg = p_ref[0] + p_ref[1]
        o_ref[...] = jnp.maximum(
            h_ref[...] + jnp.dot(agg, w_ref[...], preferred_element_type=_f32),
            0.0)

    return pl.pallas_call(
        body,
        grid=(NP // _RBN,),
        in_specs=[
            pl.BlockSpec((_RBN, H), lambda i: (i, 0)),
            pl.BlockSpec((NUM_SC, _RBN, H), lambda i: (0, i, 0)),
            pl.BlockSpec((H, H), lambda i: (0, 0)),
        ],
        out_specs=pl.BlockSpec((_RBN, H), lambda i: (i, 0)),
        out_shape=jax.ShapeDtypeStruct((NP, H), _f32),
    )(h, parts, w_h)


def _tc_bond_mlp(ga, w1, b1, w2, b2):
    nblk = NB // _RB

    def body(g0_ref, g1_ref, w1_ref, b1_ref, w2_ref, b2_ref, o_ref):
        e = (g0_ref[...].astype(_f32) + g1_ref[...].astype(_f32)).astype(_bf16)
        t = jnp.maximum(
            jnp.dot(e, w1_ref[...], preferred_element_type=_f32) + b1_ref[...],
            0.0)
        o_ref[...] = jnp.dot(t, w2_ref[...], preferred_element_type=_f32) + b2_ref[...]

    return pl.pallas_call(
        body,
        grid=(nblk,),
        in_specs=[
            pl.BlockSpec((_RB, H), lambda i: (i, 0)),
            pl.BlockSpec((_RB, H), lambda i: (i + nblk, 0)),
            pl.BlockSpec((H, H), lambda i: (0, 0)),
            pl.BlockSpec((1, H), lambda i: (0, 0)),
            pl.BlockSpec((H, 2), lambda i: (0, 0)),
            pl.BlockSpec((1, 2), lambda i: (0, 0)),
        ],
        out_specs=pl.BlockSpec((_RB, 2), lambda i: (i, 0)),
        out_shape=jax.ShapeDtypeStruct((NB, 2), _f32),
    )(ga, ga, w1, b1, w2, b2)


def _tc_angle_mlp(ga, w1, b1, w2, b2):
    nblk = NA // _RB
    off = 2 * (NB // _RB)

    def body(e0_ref, e2_ref, c_ref, w1_ref, b1_ref, w2_ref, b2_ref, o_ref):
        ends = (e0_ref[...].astype(_f32) + e2_ref[...].astype(_f32)).astype(_bf16)
        t = (jnp.dot(ends, w1_ref[0:H, :], preferred_element_type=_f32)
             + jnp.dot(c_ref[...].astype(_bf16), w1_ref[H:2 * H, :],
                       preferred_element_type=_f32)
             + b1_ref[...])
        t = jnp.maximum(t, 0.0)
        o_ref[...] = jnp.dot(t, w2_ref[...], preferred_element_type=_f32) + b2_ref[...]

    return pl.pallas_call(
        body,
        grid=(nblk,),
        in_specs=[
            pl.BlockSpec((_RB, H), lambda i: (i + off, 0)),
            pl.BlockSpec((_RB, H), lambda i: (i + off + nblk, 0)),
            pl.BlockSpec((_RB, H), lambda i: (i + off + 2 * nblk, 0)),
            pl.BlockSpec((2 * H, H), lambda i: (0, 0)),
            pl.BlockSpec((1, H), lambda i: (0, 0)),
            pl.BlockSpec((H, 2), lambda i: (0, 0)),
            pl.BlockSpec((1, 2), lambda i: (0, 0)),
        ],
        out_specs=pl.BlockSpec((_RB, 2), lambda i: (i, 0)),
        out_shape=jax.ShapeDtypeStruct((NA, 2), _f32),
    )(ga, ga, ga, w1, b1, w2, b2)


def _tc_dihedral_mlp(ga, w1, w1r, b1, w2, b2):
    nblk = ND // _RB
    off = 0

    def body(g0, g1, g2, g3, w1_ref, w1r_ref, b1_ref, w2_ref, b2_ref, o_ref):
        gs = (g0[...].astype(_bf16), g1[...].astype(_bf16),
              g2[...].astype(_bf16), g3[...].astype(_bf16))
        t1 = b1_ref[...]
        t2 = b1_ref[...]
        for kk in range(4):
            blk = slice(kk * H, (kk + 1) * H)
            t1 = t1 + jnp.dot(gs[kk], w1_ref[blk, :], preferred_element_type=_f32)
            t2 = t2 + jnp.dot(gs[kk], w1r_ref[blk, :], preferred_element_type=_f32)
        t = jnp.maximum(t1, 0.0) + jnp.maximum(t2, 0.0)
        o_ref[...] = (0.5 * jnp.dot(t, w2_ref[...], preferred_element_type=_f32)
                      + b2_ref[...])

    return pl.pallas_call(
        body,
        grid=(nblk,),
        in_specs=[
            pl.BlockSpec((_RB, H), lambda i: (i + off, 0)),
            pl.BlockSpec((_RB, H), lambda i: (i + off + nblk, 0)),
            pl.BlockSpec((_RB, H), lambda i: (i + off + 2 * nblk, 0)),
            pl.BlockSpec((_RB, H), lambda i: (i + off + 3 * nblk, 0)),
            pl.BlockSpec((4 * H, H), lambda i: (0, 0)),
            pl.BlockSpec((4 * H, H), lambda i: (0, 0)),
            pl.BlockSpec((1, H), lambda i: (0, 0)),
            pl.BlockSpec((H, 4), lambda i: (0, 0)),
            pl.BlockSpec((1, 4), lambda i: (0, 0)),
        ],
        out_specs=pl.BlockSpec((_RB, 4), lambda i: (i, 0)),
        out_shape=jax.ShapeDtypeStruct((ND, 4), _f32),
    )(ga, ga, ga, ga, w1, w1r, b1, w2, b2)


def _tc_pair_mlp(enc, lj, w1, b1, w2, b2):
    def body(e_ref, lj_ref, w1_ref, b1_ref, w2_ref, b2_ref, o_ref):
        t = jnp.maximum(
            jnp.dot(e_ref[...], w1_ref[...], preferred_element_type=_f32)
            + b1_ref[...], 0.0)
        tp = jnp.dot(t, w2_ref[...], preferred_element_type=_f32) + b2_ref[...]
        o_ref[...] = jnp.concatenate([tp, lj_ref[...]], axis=1)

    return pl.pallas_call(
        body,
        grid=(N // _RB,),
        in_specs=[
            pl.BlockSpec((_RB, H), lambda i: (i, 0)),
            pl.BlockSpec((_RB, 2), lambda i: (i, 0)),
            pl.BlockSpec((H, H), lambda i: (0, 0)),
            pl.BlockSpec((1, H), lambda i: (0, 0)),
            pl.BlockSpec((H, 2), lambda i: (0, 0)),
            pl.BlockSpec((1, 2), lambda i: (0, 0)),
        ],
        out_specs=pl.BlockSpec((_RB, 4), lambda i: (i, 0)),
        out_shape=jax.ShapeDtypeStruct((N, 4), _f32),
    )(enc, lj, w1, b1, w2, b2)


def kernel(x, edge_index, bonds, angles, dihedrals, lj_params,
           W_in, W_h,
           bw1, bb1, bw2, bb2,
           aw1, ab1, aw2, ab2,
           dw1, db1, dw2, db2,
           pw1, pb1, pw2, pb2):
    src = edge_index[0]
    dst = edge_index[1]
    pad_e = E_PAD - E
    src_pad = (jnp.arange(pad_e, dtype=jnp.int32) * 37) % N
    dst_pad = N + (jnp.arange(pad_e, dtype=jnp.int32) % TRASH_ROWS)
    src3 = jnp.concatenate([src, src_pad]).reshape(NW, PH_E, PCH_E, CH_E)
    dst3 = jnp.concatenate([dst, dst_pad]).reshape(NW, PH_E, PCH_E, CH_E)
    ga_pad = (jnp.arange(GA_PAD - GA_REAL, dtype=jnp.int32) * 37) % N
    idxA3 = jnp.concatenate(
        [bonds[:, 0], bonds[:, 1],
         angles[:, 0], angles[:, 2], angles[:, 1],
         ga_pad]).reshape(NW, CPW_GA, CH)
    gb_pad = (jnp.arange(GB_PAD - GB_REAL, dtype=jnp.int32) * 37) % N
    idxB3 = jnp.concatenate(
        [dihedrals[:, 0], dihedrals[:, 1], dihedrals[:, 2], dihedrals[:, 3],
         gb_pad]).reshape(NW, CPW_GB, CH)
    zeros = jnp.zeros((NP, H), _f32)
    dw1b = dw1.astype(_bf16)
    dw1r = jnp.concatenate(
        [dw1b[3 * H:4 * H], dw1b[2 * H:3 * H], dw1b[H:2 * H], dw1b[0:H]], axis=0)

    xp = jnp.pad(x, ((0, NP - N), (0, 0)))
    h = _tc_encode_init(xp, W_in)
    for _ in range(DEPTH):
        parts = _sc_round_agg(h, src3, dst3, zeros)
        h = _tc_round_update(h, parts, W_h)

    gaA = _sc_term_gather(h, idxA3, CPW_GA, GA_PAD)
    gaB = _sc_term_gather(h, idxB3, CPW_GB, GB_PAD)

    bond_params = _tc_bond_mlp(
        gaA, bw1.astype(_bf16), bb1.reshape(1, H), bw2, bb2.reshape(1, 2))
    angle_params = _tc_angle_mlp(
        gaA, aw1.astype(_bf16), ab1.reshape(1, H), aw2, ab2.reshape(1, 2))
    pair_params = _tc_pair_mlp(
        h, lj_params, pw1, pb1.reshape(1, H), pw2, pb2.reshape(1, 2))
    dihedral_params = _tc_dihedral_mlp(
        gaB, dw1b, dw1r, db1.reshape(1, H), dw2, db2.reshape(1, 4))
    return bond_params, angle_params, dihedral_params, pair_params


# speedup vs baseline: 1.0704x; 1.0541x over previous
import functools

import jax
import jax.numpy as jnp
from jax import lax
from jax.experimental import pallas as pl
from jax.experimental.pallas import tpu as pltpu
from jax.experimental.pallas import tpu_sc as plsc

N = 10000
E = 320000
D = 128
H = 128
NB = 20000
NA = 20000
ND = 30000
DEPTH = 3

NUM_SC = 2
NUM_SUBCORES = 16
NW = NUM_SC * NUM_SUBCORES
CH = 128
NBUF = 4
NBUF_E = 3

NP = 10240
TRASH_ROWS = NP - N

CH_E = 96
CPW_E = 105
PH_E = 5
PCH_E = CPW_E // PH_E
E_PAD = NW * CPW_E * CH_E

GA_REAL = 2 * NB + 3 * NA
GB_REAL = 4 * ND
NBUF_G = 5
CPW_GA = 25
CPW_GB = 30
GA_PAD = NW * CPW_GA * CH
GB_PAD = NW * CPW_GB * CH

ROWS_PER_TILE = NP // NUM_SUBCORES

_f32 = jnp.float32
_bf16 = jnp.bfloat16


def _vector_mesh():
    return plsc.VectorSubcoreMesh(core_axis_name="core", subcore_axis_name="subcore")


def _sc_round_agg(h, src3, dst3, zeros):
    @functools.partial(
        pl.kernel,
        out_type=jax.ShapeDtypeStruct((NUM_SC, NP, H), _f32),
        mesh=_vector_mesh(),
        scratch_types=[
            pltpu.VMEM((2, PCH_E, CH_E), jnp.int32),
            pltpu.VMEM((2, PCH_E, CH_E), jnp.int32),
            [pltpu.VMEM((CH_E, H), _f32) for _ in range(NBUF_E)],
            pltpu.VMEM_SHARED((NP, H), _f32),
            pltpu.SemaphoreType.DMA((NBUF_E,)),
            pltpu.SemaphoreType.DMA((NBUF_E,)),
            pltpu.SemaphoreType.DMA((2,)),
            pltpu.SemaphoreType.DMA,
        ],
    )
    def k(h_hbm, src_hbm, dst_hbm, z_hbm, out_hbm,
          src_v, dst_v, rows_v, agg_sh, sem_g, sem_s, sem_i, sem):
        cid = lax.axis_index("core")
        sid = lax.axis_index("subcore")
        wid = sid * NUM_SC + cid
        row0 = sid * ROWS_PER_TILE

        zc = pltpu.async_copy(
            z_hbm.at[pl.ds(row0, ROWS_PER_TILE)],
            agg_sh.at[pl.ds(row0, ROWS_PER_TILE)], sem)
        pltpu.async_copy(src_hbm.at[wid, 0], src_v.at[0], sem_i.at[0])
        pltpu.async_copy(dst_hbm.at[wid, 0], dst_v.at[0], sem_i.at[0])
        zc.wait()
        plsc.subcore_barrier()

        for p in range(PH_E):
            cur = p % 2
            pltpu.make_async_copy(
                src_hbm.at[wid, p], src_v.at[cur], sem_i.at[cur]).wait()
            pltpu.make_async_copy(
                dst_hbm.at[wid, p], dst_v.at[cur], sem_i.at[cur]).wait()
            if p + 1 < PH_E:
                nxt = (p + 1) % 2
                pltpu.async_copy(src_hbm.at[wid, p + 1], src_v.at[nxt],
                                 sem_i.at[nxt])
                pltpu.async_copy(dst_hbm.at[wid, p + 1], dst_v.at[nxt],
                                 sem_i.at[nxt])
            sv = src_v.at[cur]
            dv = dst_v.at[cur]

            for b in range(NBUF_E):
                pltpu.async_copy(h_hbm.at[sv.at[b]], rows_v[b], sem_g.at[b])

            @pl.loop(0, PCH_E, step=NBUF_E)
            def _(jj):
                for b in range(NBUF_E):
                    j = jj + b
                    pltpu.make_async_copy(
                        h_hbm.at[sv.at[j]], rows_v[b], sem_g.at[b]).wait()
                    pltpu.async_copy(
                        rows_v[b], agg_sh.at[dv.at[j]], sem_s.at[b], add=True)

                    @pl.when(j + NBUF_E < PCH_E)
                    def _():
                        pltpu.make_async_copy(
                            rows_v[b], agg_sh.at[dv.at[j]], sem_s.at[b]).wait()
                        pltpu.async_copy(
                            h_hbm.at[sv.at[j + NBUF_E]], rows_v[b], sem_g.at[b])

            for b in range(NBUF_E):
                j = PCH_E - NBUF_E + b
                pltpu.make_async_copy(
                    rows_v[b], agg_sh.at[dv.at[j]], sem_s.at[b]).wait()

        plsc.subcore_barrier()
        pltpu.async_copy(
            agg_sh.at[pl.ds(row0, ROWS_PER_TILE)],
            out_hbm.at[cid, pl.ds(row0, ROWS_PER_TILE)], sem).wait()

    return k(h, src3, dst3, zeros)


def _sc_term_gather(enc, idx3, cpw, out_rows):
    @functools.partial(
        pl.kernel,
        out_type=jax.ShapeDtypeStruct((out_rows, H), _f32),
        mesh=_vector_mesh(),
        scratch_types=[
            pltpu.VMEM((cpw, CH), jnp.int32),
            [pltpu.VMEM((CH, H), _f32) for _ in range(NBUF_G)],
            pltpu.SemaphoreType.DMA((NBUF_G,)),
            pltpu.SemaphoreType.DMA((NBUF_G,)),
            pltpu.SemaphoreType.DMA,
        ],
    )
    def k(enc_hbm, idx_hbm, out_hbm, idx_v, rows_v, sem_g, sem_s, sem):
        cid = lax.axis_index("core")
        sid = lax.axis_index("subcore")
        wid = sid * NUM_SC + cid
        base = wid * cpw * CH

        pltpu.async_copy(idx_hbm.at[wid], idx_v, sem).wait()

        for b in range(NBUF_G):
            pltpu.async_copy(enc_hbm.at[idx_v.at[b]], rows_v[b], sem_g.at[b])

        @pl.loop(0, cpw, step=NBUF_G)
        def _(jj):
            for b in range(NBUF_G):
                j = jj + b
                pltpu.make_async_copy(
                    enc_hbm.at[idx_v.at[j]], rows_v[b], sem_g.at[b]).wait()
                dst = out_hbm.at[pl.ds(base + j * CH, CH)]
                pltpu.async_copy(rows_v[b], dst, sem_s.at[b])

                @pl.when(j + NBUF_G < cpw)
                def _():
                    pltpu.make_async_copy(rows_v[b], dst, sem_s.at[b]).wait()
                    pltpu.async_copy(
                        enc_hbm.at[idx_v.at[j + NBUF_G]], rows_v[b], sem_g.at[b])

        for b in range(NBUF_G):
            j = cpw - NBUF_G + b
            dst = out_hbm.at[pl.ds(base + j * CH, CH)]
            pltpu.make_async_copy(rows_v[b], dst, sem_s.at[b]).wait()

    return k(enc, idx3)


_RB = 2000
_RBN = 2048


def _tc_encode_init(x, w_in):
    def body(x_ref, w_ref, o_ref):
        o_ref[...] = jnp.maximum(
            jnp.dot(x_ref[...], w_ref[...], preferred_element_type=_f32), 0.0)

    return pl.pallas_call(
        body,
        grid=(NP // _RBN,),
        in_specs=[
            pl.BlockSpec((_RBN, D), lambda i: (i, 0)),
            pl.BlockSpec((D, H), lambda i: (0, 0)),
        ],
        out_specs=pl.BlockSpec((_RBN, H), lambda i: (i, 0)),
        out_shape=jax.ShapeDtypeStruct((NP, H), _f32),
    )(x, w_in)


def _tc_round_update(h, parts, w_h):
    def body(h_ref, p_ref, w_ref, o_ref):
        agg = p_ref[0] + p_ref[1]
        o_ref[...] = jnp.maximum(
            h_ref[...] + jnp.dot(agg, w_ref[...], preferred_element_type=_f32),
            0.0)

    return pl.pallas_call(
        body,
        grid=(NP // _RBN,),
        in_specs=[
            pl.BlockSpec((_RBN, H), lambda i: (i, 0)),
            pl.BlockSpec((NUM_SC, _RBN, H), lambda i: (0, i, 0)),
            pl.BlockSpec((H, H), lambda i: (0, 0)),
        ],
        out_specs=pl.BlockSpec((_RBN, H), lambda i: (i, 0)),
        out_shape=jax.ShapeDtypeStruct((NP, H), _f32),
    )(h, parts, w_h)


def _tc_bond_mlp(ga, w1, b1, w2, b2):
    nblk = NB // _RB

    def body(g0_ref, g1_ref, w1_ref, b1_ref, w2_ref, b2_ref, o_ref):
        e = (g0_ref[...].astype(_f32) + g1_ref[...].astype(_f32)).astype(_bf16)
        t = jnp.maximum(
            jnp.dot(e, w1_ref[...], preferred_element_type=_f32) + b1_ref[...],
            0.0)
        o_ref[...] = jnp.dot(t, w2_ref[...], preferred_element_type=_f32) + b2_ref[...]

    return pl.pallas_call(
        body,
        grid=(nblk,),
        in_specs=[
            pl.BlockSpec((_RB, H), lambda i: (i, 0)),
            pl.BlockSpec((_RB, H), lambda i: (i + nblk, 0)),
            pl.BlockSpec((H, H), lambda i: (0, 0)),
            pl.BlockSpec((1, H), lambda i: (0, 0)),
            pl.BlockSpec((H, 2), lambda i: (0, 0)),
            pl.BlockSpec((1, 2), lambda i: (0, 0)),
        ],
        out_specs=pl.BlockSpec((_RB, 2), lambda i: (i, 0)),
        out_shape=jax.ShapeDtypeStruct((NB, 2), _f32),
    )(ga, ga, w1, b1, w2, b2)


def _tc_angle_mlp(ga, w1, b1, w2, b2):
    nblk = NA // _RB
    off = 2 * (NB // _RB)

    def body(e0_ref, e2_ref, c_ref, w1_ref, b1_ref, w2_ref, b2_ref, o_ref):
        ends = (e0_ref[...].astype(_f32) + e2_ref[...].astype(_f32)).astype(_bf16)
        t = (jnp.dot(ends, w1_ref[0:H, :], preferred_element_type=_f32)
             + jnp.dot(c_ref[...].astype(_bf16), w1_ref[H:2 * H, :],
                       preferred_element_type=_f32)
             + b1_ref[...])
        t = jnp.maximum(t, 0.0)
        o_ref[...] = jnp.dot(t, w2_ref[...], preferred_element_type=_f32) + b2_ref[...]

    return pl.pallas_call(
        body,
        grid=(nblk,),
        in_specs=[
            pl.BlockSpec((_RB, H), lambda i: (i + off, 0)),
            pl.BlockSpec((_RB, H), lambda i: (i + off + nblk, 0)),
            pl.BlockSpec((_RB, H), lambda i: (i + off + 2 * nblk, 0)),
            pl.BlockSpec((2 * H, H), lambda i: (0, 0)),
            pl.BlockSpec((1, H), lambda i: (0, 0)),
            pl.BlockSpec((H, 2), lambda i: (0, 0)),
            pl.BlockSpec((1, 2), lambda i: (0, 0)),
        ],
        out_specs=pl.BlockSpec((_RB, 2), lambda i: (i, 0)),
        out_shape=jax.ShapeDtypeStruct((NA, 2), _f32),
    )(ga, ga, ga, w1, b1, w2, b2)


def _tc_dihedral_mlp(ga, w1, w1r, b1, w2, b2):
    nblk = ND // _RB
    off = 0

    def body(g0, g1, g2, g3, w1_ref, w1r_ref, b1_ref, w2_ref, b2_ref, o_ref):
        gs = (g0[...].astype(_bf16), g1[...].astype(_bf16),
              g2[...].astype(_bf16), g3[...].astype(_bf16))
        t1 = b1_ref[...]
        t2 = b1_ref[...]
        for kk in range(4):
            blk = slice(kk * H, (kk + 1) * H)
            t1 = t1 + jnp.dot(gs[kk], w1_ref[blk, :], preferred_element_type=_f32)
            t2 = t2 + jnp.dot(gs[kk], w1r_ref[blk, :], preferred_element_type=_f32)
        t = jnp.maximum(t1, 0.0) + jnp.maximum(t2, 0.0)
        o_ref[...] = (0.5 * jnp.dot(t, w2_ref[...], preferred_element_type=_f32)
                      + b2_ref[...])

    return pl.pallas_call(
        body,
        grid=(nblk,),
        in_specs=[
            pl.BlockSpec((_RB, H), lambda i: (i + off, 0)),
            pl.BlockSpec((_RB, H), lambda i: (i + off + nblk, 0)),
            pl.BlockSpec((_RB, H), lambda i: (i + off + 2 * nblk, 0)),
            pl.BlockSpec((_RB, H), lambda i: (i + off + 3 * nblk, 0)),
            pl.BlockSpec((4 * H, H), lambda i: (0, 0)),
            pl.BlockSpec((4 * H, H), lambda i: (0, 0)),
            pl.BlockSpec((1, H), lambda i: (0, 0)),
            pl.BlockSpec((H, 4), lambda i: (0, 0)),
            pl.BlockSpec((1, 4), lambda i: (0, 0)),
        ],
        out_specs=pl.BlockSpec((_RB, 4), lambda i: (i, 0)),
        out_shape=jax.ShapeDtypeStruct((ND, 4), _f32),
    )(ga, ga, ga, ga, w1, w1r, b1, w2, b2)


def _tc_pair_mlp(enc, lj, w1, b1, w2, b2):
    def body(e_ref, lj_ref, w1_ref, b1_ref, w2_ref, b2_ref, o_ref):
        t = jnp.maximum(
            jnp.dot(e_ref[...], w1_ref[...], preferred_element_type=_f32)
            + b1_ref[...], 0.0)
        tp = jnp.dot(t, w2_ref[...], preferred_element_type=_f32) + b2_ref[...]
        o_ref[...] = jnp.concatenate([tp, lj_ref[...]], axis=1)

    return pl.pallas_call(
        body,
        grid=(N // _RB,),
        in_specs=[
            pl.BlockSpec((_RB, H), lambda i: (i, 0)),
            pl.BlockSpec((_RB, 2), lambda i: (i, 0)),
            pl.BlockSpec((H, H), lambda i: (0, 0)),
            pl.BlockSpec((1, H), lambda i: (0, 0)),
            pl.BlockSpec((H, 2), lambda i: (0, 0)),
            pl.BlockSpec((1, 2), lambda i: (0, 0)),
        ],
        out_specs=pl.BlockSpec((_RB, 4), lambda i: (i, 0)),
        out_shape=jax.ShapeDtypeStruct((N, 4), _f32),
    )(enc, lj, w1, b1, w2, b2)


def kernel(x, edge_index, bonds, angles, dihedrals, lj_params,
           W_in, W_h,
           bw1, bb1, bw2, bb2,
           aw1, ab1, aw2, ab2,
           dw1, db1, dw2, db2,
           pw1, pb1, pw2, pb2):
    src = edge_index[0]
    dst = edge_index[1]
    pad_e = E_PAD - E
    src_pad = (jnp.arange(pad_e, dtype=jnp.int32) * 37) % N
    dst_pad = N + (jnp.arange(pad_e, dtype=jnp.int32) % TRASH_ROWS)
    src3 = jnp.concatenate([src, src_pad]).reshape(NW, PH_E, PCH_E, CH_E)
    dst3 = jnp.concatenate([dst, dst_pad]).reshape(NW, PH_E, PCH_E, CH_E)
    ga_pad = (jnp.arange(GA_PAD - GA_REAL, dtype=jnp.int32) * 37) % N
    idxA3 = jnp.concatenate(
        [bonds[:, 0], bonds[:, 1],
         angles[:, 0], angles[:, 2], angles[:, 1],
         ga_pad]).reshape(NW, CPW_GA, CH)
    gb_pad = (jnp.arange(GB_PAD - GB_REAL, dtype=jnp.int32) * 37) % N
    idxB3 = jnp.concatenate(
        [dihedrals[:, 0], dihedrals[:, 1], dihedrals[:, 2], dihedrals[:, 3],
         gb_pad]).reshape(NW, CPW_GB, CH)
    zeros = jnp.zeros((NP, H), _f32)
    dw1b = dw1.astype(_bf16)
    dw1r = jnp.concatenate(
        [dw1b[3 * H:4 * H], dw1b[2 * H:3 * H], dw1b[H:2 * H], dw1b[0:H]], axis=0)

    xp = jnp.pad(x, ((0, NP - N), (0, 0)))
    h = _tc_encode_init(xp, W_in)
    for _ in range(DEPTH):
        parts = _sc_round_agg(h, src3, dst3, zeros)
        h = _tc_round_update(h, parts, W_h)

    gaA = _sc_term_gather(h, idxA3, CPW_GA, GA_PAD)
    gaB = _sc_term_gather(h, idxB3, CPW_GB, GB_PAD)

    bond_params = _tc_bond_mlp(
        gaA, bw1.astype(_bf16), bb1.reshape(1, H), bw2, bb2.reshape(1, 2))
    angle_params = _tc_angle_mlp(
        gaA, aw1.astype(_bf16), ab1.reshape(1, H), aw2, ab2.reshape(1, 2))
    pair_params = _tc_pair_mlp(
        h, lj_params, pw1, pb1.reshape(1, H), pw2, pb2.reshape(1, 2))
    dihedral_params = _tc_dihedral_mlp(
        gaB, dw1b, dw1r, db1.reshape(1, H), dw2, db2.reshape(1, 4))
    return bond_params, angle_params, dihedral_params, pair_params
